# baseline (speedup 1.0000x reference)
import functools

import jax
import jax.numpy as jnp
from jax import lax
from jax.experimental import pallas as pl
from jax.experimental.pallas import tpu as pltpu
from jax.experimental.pallas import tpu_sc as plsc

N_USERS = 5000
N_NODES = 10000
D = 128
E = 320000
B = 4096
N_HOPS = 3

NC = 2
NS = 16
L = 16
NW = NC * NS
CHUNK = 128
NCHUNKS = 80
EPW = NCHUNKS * CHUNK
E_PAD = NW * EPW
TOTC = NW * NCHUNKS
K0 = 156
K1 = TOTC // NS - K0
N_PAD = 10240
ROWS_PER_TILE = N_PAD // NS
ZROWS = 128
BPW = B // NW
NG = D // L

_mesh = plsc.VectorSubcoreMesh(
    core_axis_name="c", subcore_axis_name="s", num_cores=NC, num_subcores=NS
)


def _make_hop():
    def body(tbl, epack, out, acc, eb0, eb1, eb2, eb3, rows0, rows1,
             sg0, sg1, st0, st1, st2, st3):
        cid = lax.axis_index("c")
        sid = lax.axis_index("s")
        wid = sid * NC + cid
        ebufs = (eb0, eb1, eb2, eb3)
        rows = (rows0, rows1)
        sems = (sg0, sg1)
        stgs = (st0, st1, st2, st3)

        def _zrow(r, _):
            for g in range(NG):
                rows0[r, pl.ds(g * L, L)] = jnp.zeros((L,), jnp.float32)
            return 0

        lax.fori_loop(0, ZROWS, _zrow, 0)
        for k in range(ROWS_PER_TILE // ZROWS):
            pltpu.sync_copy(
                rows0, acc.at[pl.ds(sid * ROWS_PER_TILE + k * ZROWS, ZROWS)]
            )
        plsc.subcore_barrier()

        nmine = jnp.where(cid == 0, K0, K1)
        base = jnp.where(cid == 0, sid * K0, NS * K0 + sid * K1)
        limit = base + nmine
        for b in range(2):
            pltpu.sync_copy(epack.at[base + b], ebufs[b])
        for b in (2, 3):
            pltpu.async_copy(epack.at[base + b], ebufs[b], stgs[b])
        for b in range(2):
            pltpu.async_copy(tbl.at[ebufs[b].at[0]], rows[b], sems[b])

        def chunk_quad(j, _):
            for q in range(4):
                c = base + 4 * j + q
                rb = q % 2
                pltpu.make_async_copy(
                    tbl.at[ebufs[q].at[0]], rows[rb], sems[rb]
                ).wait()

                def scale(eb, _):
                    w16 = lax.bitcast_convert_type(ebufs[q][2, pl.ds(eb * L, L)], jnp.float32)
                    for e16 in range(L):
                        e = eb * L + e16
                        w = w16[e16]
                        for g in range(NG):
                            rows[rb][e, pl.ds(g * L, L)] = (
                                rows[rb][e, pl.ds(g * L, L)] * w
                            )
                    return 0

                lax.fori_loop(0, CHUNK // L, scale, 0)
                pltpu.sync_copy(rows[rb], acc.at[ebufs[q].at[1]], add=True)

                @pl.when(c + 2 < limit)
                def _():
                    pltpu.make_async_copy(
                        epack.at[c + 2], ebufs[(q + 2) % 4], stgs[(q + 2) % 4]
                    ).wait()
                    pltpu.async_copy(
                        tbl.at[ebufs[(q + 2) % 4].at[0]], rows[rb], sems[rb]
                    )

                @pl.when(c + 4 < limit)
                def _():
                    pltpu.async_copy(epack.at[c + 4], ebufs[q], stgs[q])

            return 0

        lax.fori_loop(0, nmine // 4, chunk_quad, 0)
        plsc.subcore_barrier()

        for k in range(ROWS_PER_TILE // ZROWS):
            off = sid * ROWS_PER_TILE + k * ZROWS
            pltpu.sync_copy(acc.at[pl.ds(off, ZROWS)], rows0)
            pltpu.sync_copy(rows0, out.at[cid, pl.ds(off, ZROWS)])

    return pl.kernel(
        body,
        out_type=jax.ShapeDtypeStruct((NC, N_PAD, D), jnp.float32),
        mesh=_mesh,
        scratch_types=[
            pltpu.VMEM_SHARED((N_PAD, D), jnp.float32),
            pltpu.VMEM((8, CHUNK), jnp.int32),
            pltpu.VMEM((8, CHUNK), jnp.int32),
            pltpu.VMEM((8, CHUNK), jnp.int32),
            pltpu.VMEM((8, CHUNK), jnp.int32),
            pltpu.VMEM((CHUNK, D), jnp.float32),
            pltpu.VMEM((CHUNK, D), jnp.float32),
            pltpu.SemaphoreType.DMA,
            pltpu.SemaphoreType.DMA,
            pltpu.SemaphoreType.DMA,
            pltpu.SemaphoreType.DMA,
            pltpu.SemaphoreType.DMA,
            pltpu.SemaphoreType.DMA,
        ],
    )


def _make_tail():
    def body(emb, t1, t2, p3, users2, pos2, neg2,
             out_u, out_p, out_n, out_ru, out_rp, out_rn,
             idx_v, raw_v, acc_v, tmp_v, sem):
        cid = lax.axis_index("c")
        sid = lax.axis_index("s")
        wid = sid * NC + cid

        for idx_hbm, out_pool, out_raw in (
            (users2, out_u, out_ru),
            (pos2, out_p, out_rp),
            (neg2, out_n, out_rn),
        ):
            pltpu.sync_copy(idx_hbm.at[wid], idx_v)
            pltpu.async_copy(emb.at[idx_v], raw_v, sem).wait()
            pltpu.sync_copy(raw_v, out_raw.at[pl.ds(wid * BPW, BPW)])
            pltpu.async_copy(emb.at[idx_v], acc_v, sem).wait()
            for tbl in (t1, t2, p3.at[0], p3.at[1]):
                pltpu.async_copy(tbl.at[idx_v], tmp_v, sem).wait()

                def add(e, _):
                    for g in range(NG):
                        s = (e, pl.ds(g * L, L))
                        acc_v[s] = acc_v[s] + tmp_v[s]
                    return 0

                lax.fori_loop(0, BPW, add, 0)

            quarter = jnp.float32(0.25)

            def scl(e, _):
                for g in range(NG):
                    s = (e, pl.ds(g * L, L))
                    acc_v[s] = acc_v[s] * quarter
                return 0

            lax.fori_loop(0, BPW, scl, 0)
            pltpu.sync_copy(acc_v, out_pool.at[pl.ds(wid * BPW, BPW)])

    shp = jax.ShapeDtypeStruct((B, D), jnp.float32)
    return pl.kernel(
        body,
        out_type=(shp, shp, shp, shp, shp, shp),
        mesh=_mesh,
        scratch_types=[
            pltpu.VMEM((BPW,), jnp.int32),
            pltpu.VMEM((BPW, D), jnp.float32),
            pltpu.VMEM((BPW, D), jnp.float32),
            pltpu.VMEM((BPW, D), jnp.float32),
            pltpu.SemaphoreType.DMA,
        ],
    )


_hop = _make_hop()
_tail = _make_tail()


def kernel(embeds, edge_weight, edge_index, users, pos_items, neg_items):
    pad_idx = jnp.zeros((2, E_PAD - E), jnp.int32)
    pad_w = jnp.zeros((E_PAD - E,), jnp.float32)
    eidx = jnp.concatenate([edge_index, pad_idx], axis=1)
    ew = jnp.concatenate([edge_weight, pad_w], axis=0)
    ridx3 = eidx[0].reshape(NW, NCHUNKS, 1, CHUNK)
    cidx3 = eidx[1].reshape(NW, NCHUNKS, 1, CHUNK)
    w3 = jax.lax.bitcast_convert_type(ew, jnp.int32).reshape(NW, NCHUNKS, 1, CHUNK)
    zpad = jnp.zeros((NW, NCHUNKS, 5, CHUNK), jnp.int32)
    epack = jnp.concatenate([ridx3, cidx3, w3, zpad], axis=2).reshape(
        TOTC, 8, CHUNK
    )

    emb_pad = jnp.concatenate(
        [embeds, jnp.zeros((N_PAD - N_NODES, D), jnp.float32)], axis=0
    )
    p1 = _hop(emb_pad, epack)
    t1 = p1[0] + p1[1]
    p2 = _hop(t1, epack)
    t2 = p2[0] + p2[1]
    p3 = _hop(t2, epack)

    u2 = users.reshape(NW, BPW)
    pp2 = pos_items.reshape(NW, BPW)
    nn2 = neg_items.reshape(NW, BPW)
    return _tail(embeds, t1, t2, p3, u2, pp2, nn2)


# speedup vs baseline: 1.0465x; 1.0465x over previous
import functools

import jax
import jax.numpy as jnp
from jax import lax
from jax.experimental import pallas as pl
from jax.experimental.pallas import tpu as pltpu
from jax.experimental.pallas import tpu_sc as plsc

N_USERS = 5000
N_NODES = 10000
D = 128
E = 320000
B = 4096
N_HOPS = 3

NC = 2
NS = 16
L = 16
NW = NC * NS
CHUNK = 128
NCHUNKS = 80
EPW = NCHUNKS * CHUNK
E_PAD = NW * EPW
TOTC = NW * NCHUNKS
K0 = 152
K1 = TOTC // NS - K0
N_PAD = 10240
ROWS_PER_TILE = N_PAD // NS
ZROWS = 128
BPW = B // NW
NG = D // L

_mesh = plsc.VectorSubcoreMesh(
    core_axis_name="c", subcore_axis_name="s", num_cores=NC, num_subcores=NS
)


def _make_hop():
    def body(tbl, epack, out, acc, eb0, eb1, eb2, eb3, rows0, rows1,
             sg0, sg1, st0, st1, st2, st3):
        cid = lax.axis_index("c")
        sid = lax.axis_index("s")
        wid = sid * NC + cid
        ebufs = (eb0, eb1, eb2, eb3)
        rows = (rows0, rows1)
        sems = (sg0, sg1)
        stgs = (st0, st1, st2, st3)

        def _zrow(r, _):
            for g in range(NG):
                rows0[r, pl.ds(g * L, L)] = jnp.zeros((L,), jnp.float32)
            return 0

        lax.fori_loop(0, ZROWS, _zrow, 0)
        for k in range(ROWS_PER_TILE // ZROWS):
            pltpu.sync_copy(
                rows0, acc.at[pl.ds(sid * ROWS_PER_TILE + k * ZROWS, ZROWS)]
            )
        plsc.subcore_barrier()

        nmine = jnp.where(cid == 1, K0, K1)
        base = jnp.where(cid == 1, sid * K0, NS * K0 + sid * K1)
        limit = base + nmine
        for b in range(2):
            pltpu.sync_copy(epack.at[base + b], ebufs[b])
        for b in (2, 3):
            pltpu.async_copy(epack.at[base + b], ebufs[b], stgs[b])
        for b in range(2):
            pltpu.async_copy(tbl.at[ebufs[b].at[0]], rows[b], sems[b])

        def chunk_quad(j, _):
            for q in range(4):
                c = base + 4 * j + q
                rb = q % 2
                pltpu.make_async_copy(
                    tbl.at[ebufs[q].at[0]], rows[rb], sems[rb]
                ).wait()

                def scale(eb, _):
                    w16 = lax.bitcast_convert_type(ebufs[q][2, pl.ds(eb * L, L)], jnp.float32)
                    for e16 in range(L):
                        e = eb * L + e16
                        w = w16[e16]
                        for g in range(NG):
                            rows[rb][e, pl.ds(g * L, L)] = (
                                rows[rb][e, pl.ds(g * L, L)] * w
                            )
                    return 0

                lax.fori_loop(0, CHUNK // L, scale, 0)
                pltpu.sync_copy(rows[rb], acc.at[ebufs[q].at[1]], add=True)

                @pl.when(c + 2 < limit)
                def _():
                    pltpu.make_async_copy(
                        epack.at[c + 2], ebufs[(q + 2) % 4], stgs[(q + 2) % 4]
                    ).wait()
                    pltpu.async_copy(
                        tbl.at[ebufs[(q + 2) % 4].at[0]], rows[rb], sems[rb]
                    )

                @pl.when(c + 4 < limit)
                def _():
                    pltpu.async_copy(epack.at[c + 4], ebufs[q], stgs[q])

            return 0

        lax.fori_loop(0, nmine // 4, chunk_quad, 0)
        plsc.subcore_barrier()

        for k in range(ROWS_PER_TILE // ZROWS):
            off = sid * ROWS_PER_TILE + k * ZROWS
            pltpu.sync_copy(acc.at[pl.ds(off, ZROWS)], rows0)
            pltpu.sync_copy(rows0, out.at[cid, pl.ds(off, ZROWS)])

    return pl.kernel(
        body,
        out_type=jax.ShapeDtypeStruct((NC, N_PAD, D), jnp.float32),
        mesh=_mesh,
        scratch_types=[
            pltpu.VMEM_SHARED((N_PAD, D), jnp.float32),
            pltpu.VMEM((8, CHUNK), jnp.int32),
            pltpu.VMEM((8, CHUNK), jnp.int32),
            pltpu.VMEM((8, CHUNK), jnp.int32),
            pltpu.VMEM((8, CHUNK), jnp.int32),
            pltpu.VMEM((CHUNK, D), jnp.float32),
            pltpu.VMEM((CHUNK, D), jnp.float32),
            pltpu.SemaphoreType.DMA,
            pltpu.SemaphoreType.DMA,
            pltpu.SemaphoreType.DMA,
            pltpu.SemaphoreType.DMA,
            pltpu.SemaphoreType.DMA,
            pltpu.SemaphoreType.DMA,
        ],
    )


def _make_tail():
    def body(emb, t1, t2, p3, users2, pos2, neg2,
             out_u, out_p, out_n, out_ru, out_rp, out_rn,
             idx_v, raw_v, acc_v, tmp_v, sem):
        cid = lax.axis_index("c")
        sid = lax.axis_index("s")
        wid = sid * NC + cid

        for idx_hbm, out_pool, out_raw in (
            (users2, out_u, out_ru),
            (pos2, out_p, out_rp),
            (neg2, out_n, out_rn),
        ):
            pltpu.sync_copy(idx_hbm.at[wid], idx_v)
            pltpu.async_copy(emb.at[idx_v], raw_v, sem).wait()
            pltpu.sync_copy(raw_v, out_raw.at[pl.ds(wid * BPW, BPW)])
            pltpu.async_copy(emb.at[idx_v], acc_v, sem).wait()
            for tbl in (t1, t2, p3.at[0], p3.at[1]):
                pltpu.async_copy(tbl.at[idx_v], tmp_v, sem).wait()

                def add(e, _):
                    for g in range(NG):
                        s = (e, pl.ds(g * L, L))
                        acc_v[s] = acc_v[s] + tmp_v[s]
                    return 0

                lax.fori_loop(0, BPW, add, 0)

            quarter = jnp.float32(0.25)

            def scl(e, _):
                for g in range(NG):
                    s = (e, pl.ds(g * L, L))
                    acc_v[s] = acc_v[s] * quarter
                return 0

            lax.fori_loop(0, BPW, scl, 0)
            pltpu.sync_copy(acc_v, out_pool.at[pl.ds(wid * BPW, BPW)])

    shp = jax.ShapeDtypeStruct((B, D), jnp.float32)
    return pl.kernel(
        body,
        out_type=(shp, shp, shp, shp, shp, shp),
        mesh=_mesh,
        scratch_types=[
            pltpu.VMEM((BPW,), jnp.int32),
            pltpu.VMEM((BPW, D), jnp.float32),
            pltpu.VMEM((BPW, D), jnp.float32),
            pltpu.VMEM((BPW, D), jnp.float32),
            pltpu.SemaphoreType.DMA,
        ],
    )


_hop = _make_hop()
_tail = _make_tail()


def kernel(embeds, edge_weight, edge_index, users, pos_items, neg_items):
    pad_idx = jnp.zeros((2, E_PAD - E), jnp.int32)
    pad_w = jnp.zeros((E_PAD - E,), jnp.float32)
    eidx = jnp.concatenate([edge_index, pad_idx], axis=1)
    ew = jnp.concatenate([edge_weight, pad_w], axis=0)
    ridx3 = eidx[0].reshape(NW, NCHUNKS, 1, CHUNK)
    cidx3 = eidx[1].reshape(NW, NCHUNKS, 1, CHUNK)
    w3 = jax.lax.bitcast_convert_type(ew, jnp.int32).reshape(NW, NCHUNKS, 1, CHUNK)
    zpad = jnp.zeros((NW, NCHUNKS, 5, CHUNK), jnp.int32)
    epack = jnp.concatenate([ridx3, cidx3, w3, zpad], axis=2).reshape(
        TOTC, 8, CHUNK
    )

    emb_pad = jnp.concatenate(
        [embeds, jnp.zeros((N_PAD - N_NODES, D), jnp.float32)], axis=0
    )
    p1 = _hop(emb_pad, epack)
    t1 = p1[0] + p1[1]
    p2 = _hop(t1, epack)
    t2 = p2[0] + p2[1]
    p3 = _hop(t2, epack)

    u2 = users.reshape(NW, BPW)
    pp2 = pos_items.reshape(NW, BPW)
    nn2 = neg_items.reshape(NW, BPW)
    return _tail(embeds, t1, t2, p3, u2, pp2, nn2)
